# Initial kernel scaffold; baseline (speedup 1.0000x reference)
import jax
import jax.numpy as jnp
from jax.experimental import pallas as pl


def kernel(x, edge_index, edge_weight, W, b):
    raise NotImplementedError("write your pallas kernel here")


# speedup vs baseline: 5.0046x; 5.0046x over previous
import functools

import jax
import jax.numpy as jnp
from jax import lax
from jax.experimental import pallas as pl
from jax.experimental.pallas import tpu as pltpu
from jax.experimental.pallas import tpu_sc as plsc

N = 10000
E = 320000
D = 128

CHUNK = 128
NUM_CHUNKS = E // CHUNK
NC, NS = 2, 16
NW = NC * NS
ITERS = (NUM_CHUNKS + NW - 1) // NW
ZCHUNK = 80
NZ = N // ZCHUNK
ZITERS = (NZ + NS - 1) // NS


def _mm_kernel(x_ref, w_ref, b_ref, o_ref):
    o_ref[...] = (
        jnp.dot(x_ref[...], w_ref[...], preferred_element_type=jnp.float32)
        + b_ref[...]
    )


def _matmul(x, W, b):
    BN = 2000
    return pl.pallas_call(
        _mm_kernel,
        grid=(N // BN,),
        in_specs=[
            pl.BlockSpec((BN, D), lambda i: (i, 0)),
            pl.BlockSpec((D, D), lambda i: (0, 0)),
            pl.BlockSpec((1, D), lambda i: (0, 0)),
        ],
        out_specs=pl.BlockSpec((BN, D), lambda i: (i, 0)),
        out_shape=jax.ShapeDtypeStruct((N, D), jnp.float32),
    )(x, W, b.reshape(1, D))


def _ep_kernel(p_ref, x_ref, o_ref):
    o_ref[...] = jnp.maximum(p_ref[0] + p_ref[1], 0.0) + x_ref[...]


def _epilogue(partial, x):
    BN = 2000
    return pl.pallas_call(
        _ep_kernel,
        grid=(N // BN,),
        in_specs=[
            pl.BlockSpec((2, BN, D), lambda i: (0, i, 0)),
            pl.BlockSpec((BN, D), lambda i: (i, 0)),
        ],
        out_specs=pl.BlockSpec((BN, D), lambda i: (i, 0)),
        out_shape=jax.ShapeDtypeStruct((N, D), jnp.float32),
    )(partial, x)


def _edge_body(hidden_hbm, src_hbm, dst_hbm, w_hbm, partial_hbm,
               idx_src, idx_dst, wbuf, rows, acc, sem):
    cid = lax.axis_index("c")
    sid = lax.axis_index("s")
    wid = sid * NC + cid

    zero = jnp.zeros((16,), jnp.float32)

    def zrow(r, carry):
        for j in range(D // 16):
            rows[r, pl.ds(16 * j, 16)] = zero
        return carry

    lax.fori_loop(0, ZCHUNK, zrow, 0)
    for k in range(ZITERS):
        c = sid + NS * k

        @pl.when(c < NZ)
        def _():
            pltpu.sync_copy(
                rows.at[pl.ds(0, ZCHUNK)],
                acc.at[pl.ds(c * ZCHUNK, ZCHUNK)],
            )

    plsc.subcore_barrier()

    def body(i, carry):
        chunk = wid + NW * i

        @pl.when(chunk < NUM_CHUNKS)
        def _():
            base = chunk * CHUNK
            pltpu.sync_copy(src_hbm.at[pl.ds(base, CHUNK)], idx_src)
            pltpu.sync_copy(dst_hbm.at[pl.ds(base, CHUNK)], idx_dst.at[0])
            pltpu.sync_copy(w_hbm.at[pl.ds(base, CHUNK)], wbuf)
            pltpu.async_copy(hidden_hbm.at[idx_src], rows, sem).wait()

            def scale(g, c2):
                w16 = wbuf[pl.ds(g * 16, 16)]
                for i in range(16):
                    s = w16[i]
                    r = g * 16 + i
                    for j in range(D // 16):
                        sl = pl.ds(16 * j, 16)
                        rows[r, sl] = rows[r, sl] * s
                return c2

            lax.fori_loop(0, CHUNK // 16, scale, 0)
            pltpu.sync_copy(rows, acc.at[idx_dst.at[0]], add=True)

        return carry

    lax.fori_loop(0, ITERS, body, 0)
    plsc.subcore_barrier()

    for k in range(ZITERS):
        c = sid + NS * k

        @pl.when(c < NZ)
        def _():
            pltpu.sync_copy(
                acc.at[pl.ds(c * ZCHUNK, ZCHUNK)],
                partial_hbm.at[cid, pl.ds(c * ZCHUNK, ZCHUNK)],
            )


def _edge_pass(hidden, src, dst, w):
    mesh = plsc.VectorSubcoreMesh(core_axis_name="c", subcore_axis_name="s")
    f = functools.partial(
        pl.kernel,
        mesh=mesh,
        out_type=jax.ShapeDtypeStruct((NC, N, D), jnp.float32),
        scratch_types=[
            pltpu.VMEM((CHUNK,), jnp.int32),
            pltpu.VMEM((1, CHUNK), jnp.int32),
            pltpu.VMEM((CHUNK,), jnp.float32),
            pltpu.VMEM((CHUNK, D), jnp.float32),
            pltpu.VMEM_SHARED((N, D), jnp.float32),
            pltpu.SemaphoreType.DMA,
        ],
    )(_edge_body)
    return f(hidden, src, dst, w)


def kernel(x, edge_index, edge_weight, W, b):
    hidden = _matmul(x, W, b)
    src = edge_index[0]
    dst = edge_index[1]
    partial = _edge_pass(hidden, src, dst, edge_weight)
    return _epilogue(partial, x)
